# baseline (speedup 1.0000x reference)
import functools

import jax
import jax.numpy as jnp
from jax import lax
from jax.experimental import pallas as pl
from jax.experimental.pallas import tpu as pltpu
from jax.experimental.pallas import tpu_sc as plsc

M = 1000000
D = 64
B = 16384
NW = 32
R = (M // NW) & ~7
TAIL_BASE = NW * R
TAIL = M - TAIL_BASE
NCHUNK = B // 16
CH = 128
NSEG = B // CH
DP = 128

_mesh = plsc.VectorSubcoreMesh(core_axis_name="c", subcore_axis_name="s")


def _sc_body(out_hbm, val_hbm, idx_hbm,
             idx_v, winner_v, pos2d_v, rows2d_v, buf_v,
             gat_sem, sct_sem):
  wid = lax.axis_index("s") * 2 + lax.axis_index("c")
  base = (wid * R).astype(jnp.int32)
  is_w0 = wid == 0

  pltpu.sync_copy(idx_hbm, idx_v)

  lanes = lax.iota(jnp.int32, 16)
  zeros16 = jnp.zeros((16,), jnp.int32)
  ones16 = jnp.ones((16,), jnp.int32)

  def owned(iv):
    local = iv - base
    intail = (iv >= TAIL_BASE) & is_w0
    inb = ((iv >= base) & (local < R)) | intail
    local = jnp.where(intail, R + (iv - TAIL_BASE), local)
    return inb, local

  def phase1(c, carry):
    pos = (c * 16).astype(jnp.int32) + lanes
    iv = idx_v[pl.ds(pl.multiple_of(c * 16, 16), 16)]
    inb, local = owned(iv)
    _, lastm = plsc.scan_count(iv, inb)
    plsc.store_scatter(winner_v, [local], pos, mask=lastm)
    return carry

  lax.fori_loop(0, NCHUNK, phase1, 0)

  def phase2(c, n_vec):
    pos = (c * 16).astype(jnp.int32) + lanes
    iv = idx_v[pl.ds(pl.multiple_of(c * 16, 16), 16)]
    inb, local = owned(iv)
    win = plsc.load_gather(winner_v, [local], mask=inb)
    keep = inb & (win == pos)
    off = n_vec + plsc.cumsum(ones16, mask=keep) - 1
    plsc.store_scatter(pos2d_v, [off >> 7, off & 127], pos, mask=keep)
    plsc.store_scatter(rows2d_v, [off >> 7, off & 127], iv, mask=keep)
    return n_vec + plsc.all_reduce_population_count(keep)

  n_vec = lax.fori_loop(0, NCHUNK, phase2, zeros16)
  n = jnp.max(n_vec)
  n_pad = ((n + CH - 1) >> 7) << 7

  first_pos = pos2d_v[0, pl.ds(0, 16)]
  first_row = rows2d_v[0, pl.ds(0, 16)]
  neg = jnp.full((16,), jnp.int32(-2**31))
  pad_pos = jnp.full((16,), jnp.max(jnp.where(lanes == 0, first_pos, neg)))
  pad_row = jnp.full((16,), jnp.max(jnp.where(lanes == 0, first_row, neg)))

  def padfill(t, carry):
    g = t * 16 + lanes
    mp = g >= n
    plsc.store_scatter(pos2d_v, [g >> 7, g & 127], pad_pos, mask=mp)
    plsc.store_scatter(rows2d_v, [g >> 7, g & 127], pad_row, mask=mp)
    return carry

  lax.fori_loop(n >> 4, n_pad >> 4, padfill, 0)

  def phase3(g, carry):
    pltpu.async_copy(val_hbm.at[pos2d_v.at[g]], buf_v, gat_sem).wait()
    pltpu.async_copy(buf_v, out_hbm.at[rows2d_v.at[g]], sct_sem).wait()
    return carry

  lax.fori_loop(0, n_pad >> 7, phase3, 0)


_sc_kernel = functools.partial(
    pl.kernel,
    out_type=(),
    mesh=_mesh,
    compiler_params=pltpu.CompilerParams(needs_layout_passes=False,
                                         use_tc_tiling_on_sc=True),
    scratch_types=[
        pltpu.VMEM((B,), jnp.int32),
        pltpu.VMEM((R + TAIL,), jnp.int32),
        pltpu.VMEM((NSEG, CH), jnp.int32),
        pltpu.VMEM((NSEG, CH), jnp.int32),
        pltpu.VMEM((CH, DP), jnp.float32),
        pltpu.SemaphoreType.DMA,
        pltpu.SemaphoreType.DMA,
    ],
)(_sc_body)


WIN = 8192


def _tin_body(inT_ref, out_ref):
  x = inT_ref[...]
  xt = jnp.swapaxes(x, 0, 1)
  out_ref[...] = jnp.concatenate(
      [xt, jnp.zeros((WIN, DP - D), jnp.float32)], axis=1)


_tin = pl.pallas_call(
    _tin_body,
    grid=((M + WIN - 1) // WIN,),
    in_specs=[pl.BlockSpec((D, WIN), lambda i: (0, i))],
    out_specs=pl.BlockSpec((WIN, DP), lambda i: (i, 0)),
    out_shape=jax.ShapeDtypeStruct((M, DP), jnp.float32),
)


def _tout_body(in_ref, outT_ref):
  outT_ref[...] = jnp.swapaxes(in_ref[:, :D], 0, 1)


_tout = pl.pallas_call(
    _tout_body,
    grid=((M + WIN - 1) // WIN,),
    in_specs=[pl.BlockSpec((WIN, DP), lambda i: (i, 0))],
    out_specs=pl.BlockSpec((D, WIN), lambda i: (0, i)),
    out_shape=jax.ShapeDtypeStruct((D, M), jnp.float32),
)


def kernel(mem, val, idx):
  mem_pad = _tin(mem.T)
  val_pad = jnp.concatenate(
      [val, jnp.zeros((B, DP - D), jnp.float32)], axis=1)
  out_ref = jax.new_ref(mem_pad)
  _sc_kernel(out_ref, val_pad, idx.astype(jnp.int32))
  return _tout(out_ref[...]).T


# speedup vs baseline: 1.0674x; 1.0674x over previous
import functools

import jax
import jax.numpy as jnp
from jax import lax
from jax.experimental import pallas as pl
from jax.experimental.pallas import tpu as pltpu
from jax.experimental.pallas import tpu_sc as plsc

M = 1000000
D = 64
B = 16384
NW = 32
R = (M // NW) & ~7
TAIL_BASE = NW * R
TAIL = M - TAIL_BASE
NCHUNK = B // 16
CH = 128
NSEG = B // CH
DP = 128

_mesh = plsc.VectorSubcoreMesh(core_axis_name="c", subcore_axis_name="s")


def _sc_body(out_hbm, val_hbm, idx_hbm,
             idx_v, winner_v, pos2d_v, rows2d_v, buf_v,
             gat_sem, sct_sem):
  wid = lax.axis_index("s") * 2 + lax.axis_index("c")
  base = (wid * R).astype(jnp.int32)
  is_w0 = wid == 0

  pltpu.sync_copy(idx_hbm, idx_v)

  lanes = lax.iota(jnp.int32, 16)
  zeros16 = jnp.zeros((16,), jnp.int32)
  ones16 = jnp.ones((16,), jnp.int32)

  def owned(iv):
    local = iv - base
    intail = (iv >= TAIL_BASE) & is_w0
    inb = ((iv >= base) & (local < R)) | intail
    local = jnp.where(intail, R + (iv - TAIL_BASE), local)
    return inb, local

  def phase1(c, carry):
    pos = (c * 16).astype(jnp.int32) + lanes
    iv = idx_v[pl.ds(pl.multiple_of(c * 16, 16), 16)]
    inb, local = owned(iv)
    _, lastm = plsc.scan_count(iv, inb)
    plsc.store_scatter(winner_v, [local], pos, mask=lastm)
    return carry

  lax.fori_loop(0, NCHUNK, phase1, 0)

  def phase2(c, n_vec):
    pos = (c * 16).astype(jnp.int32) + lanes
    iv = idx_v[pl.ds(pl.multiple_of(c * 16, 16), 16)]
    inb, local = owned(iv)
    win = plsc.load_gather(winner_v, [local], mask=inb)
    keep = inb & (win == pos)
    off = n_vec + plsc.cumsum(ones16, mask=keep) - 1
    plsc.store_scatter(pos2d_v, [off >> 7, off & 127], pos, mask=keep)
    plsc.store_scatter(rows2d_v, [off >> 7, off & 127], iv, mask=keep)
    return n_vec + plsc.all_reduce_population_count(keep)

  n_vec = lax.fori_loop(0, NCHUNK, phase2, zeros16)
  n = jnp.max(n_vec)
  n_pad = ((n + CH - 1) >> 7) << 7

  first_pos = pos2d_v[0, pl.ds(0, 16)]
  first_row = rows2d_v[0, pl.ds(0, 16)]
  neg = jnp.full((16,), jnp.int32(-2**31))
  pad_pos = jnp.full((16,), jnp.max(jnp.where(lanes == 0, first_pos, neg)))
  pad_row = jnp.full((16,), jnp.max(jnp.where(lanes == 0, first_row, neg)))

  def padfill(t, carry):
    g = t * 16 + lanes
    mp = g >= n
    plsc.store_scatter(pos2d_v, [g >> 7, g & 127], pad_pos, mask=mp)
    plsc.store_scatter(rows2d_v, [g >> 7, g & 127], pad_row, mask=mp)
    return carry

  lax.fori_loop(n >> 4, n_pad >> 4, padfill, 0)

  def phase3(g, carry):
    pltpu.async_copy(val_hbm.at[pos2d_v.at[g]], buf_v, gat_sem).wait()
    pltpu.async_copy(buf_v, out_hbm.at[rows2d_v.at[g]], sct_sem).wait()
    return carry

  lax.fori_loop(0, n_pad >> 7, phase3, 0)


_sc_kernel = functools.partial(
    pl.kernel,
    out_type=(),
    mesh=_mesh,
    compiler_params=pltpu.CompilerParams(needs_layout_passes=False,
                                         use_tc_tiling_on_sc=True),
    scratch_types=[
        pltpu.VMEM((B,), jnp.int32),
        pltpu.VMEM((R + TAIL,), jnp.int32),
        pltpu.VMEM((NSEG, CH), jnp.int32),
        pltpu.VMEM((NSEG, CH), jnp.int32),
        pltpu.VMEM((CH, DP), jnp.float32),
        pltpu.SemaphoreType.DMA,
        pltpu.SemaphoreType.DMA,
    ],
)(_sc_body)


WIN = 16384


def _tin_body(inT_ref, out_ref):
  x = inT_ref[...]
  xt = jnp.swapaxes(x, 0, 1)
  out_ref[...] = jnp.concatenate(
      [xt, jnp.zeros((WIN, DP - D), jnp.float32)], axis=1)


_tin = pl.pallas_call(
    _tin_body,
    grid=((M + WIN - 1) // WIN,),
    in_specs=[pl.BlockSpec((D, WIN), lambda i: (0, i))],
    out_specs=pl.BlockSpec((WIN, DP), lambda i: (i, 0)),
    out_shape=jax.ShapeDtypeStruct((M, DP), jnp.float32),
)


def _tout_body(in_ref, outT_ref):
  outT_ref[...] = jnp.swapaxes(in_ref[:, :D], 0, 1)


_tout = pl.pallas_call(
    _tout_body,
    grid=((M + WIN - 1) // WIN,),
    in_specs=[pl.BlockSpec((WIN, DP), lambda i: (i, 0))],
    out_specs=pl.BlockSpec((D, WIN), lambda i: (0, i)),
    out_shape=jax.ShapeDtypeStruct((D, M), jnp.float32),
)


def kernel(mem, val, idx):
  mem_pad = _tin(mem.T)
  val_pad = jnp.concatenate(
      [val, jnp.zeros((B, DP - D), jnp.float32)], axis=1)
  out_ref = jax.new_ref(mem_pad)
  _sc_kernel(out_ref, val_pad, idx.astype(jnp.int32))
  return _tout(out_ref[...]).T


# speedup vs baseline: 1.0833x; 1.0149x over previous
import functools

import jax
import jax.numpy as jnp
from jax import lax
from jax.experimental import pallas as pl
from jax.experimental.pallas import tpu as pltpu
from jax.experimental.pallas import tpu_sc as plsc

M = 1000000
D = 64
B = 16384
NW = 32
R = (M // NW) & ~7
TAIL_BASE = NW * R
TAIL = M - TAIL_BASE
NCHUNK = B // 16
CH = 128
NSEG = B // CH
DP = 128

_mesh = plsc.VectorSubcoreMesh(core_axis_name="c", subcore_axis_name="s")


def _sc_body(out_hbm, val_hbm, idx_hbm,
             idx_v, winner_v, pos2d_v, rows2d_v, buf_v,
             gat_sem, sct_sem):
  wid = lax.axis_index("s") * 2 + lax.axis_index("c")
  base = (wid * R).astype(jnp.int32)
  is_w0 = wid == 0

  pltpu.sync_copy(idx_hbm, idx_v)

  lanes = lax.iota(jnp.int32, 16)
  zeros16 = jnp.zeros((16,), jnp.int32)
  ones16 = jnp.ones((16,), jnp.int32)

  def owned(iv):
    local = iv - base
    intail = (iv >= TAIL_BASE) & is_w0
    inb = ((iv >= base) & (local < R)) | intail
    local = jnp.where(intail, R + (iv - TAIL_BASE), local)
    return inb, local

  def phase1(c, carry):
    pos = (c * 16).astype(jnp.int32) + lanes
    iv = idx_v[pl.ds(pl.multiple_of(c * 16, 16), 16)]
    inb, local = owned(iv)
    _, lastm = plsc.scan_count(iv, inb)
    plsc.store_scatter(winner_v, [local], pos, mask=lastm)
    return carry

  lax.fori_loop(0, NCHUNK, phase1, 0)

  def phase2(c, n_vec):
    pos = (c * 16).astype(jnp.int32) + lanes
    iv = idx_v[pl.ds(pl.multiple_of(c * 16, 16), 16)]
    inb, local = owned(iv)
    win = plsc.load_gather(winner_v, [local], mask=inb)
    keep = inb & (win == pos)
    off = n_vec + plsc.cumsum(ones16, mask=keep) - 1
    plsc.store_scatter(pos2d_v, [off >> 7, off & 127], pos, mask=keep)
    plsc.store_scatter(rows2d_v, [off >> 7, off & 127], iv, mask=keep)
    return n_vec + plsc.all_reduce_population_count(keep)

  n_vec = lax.fori_loop(0, NCHUNK, phase2, zeros16)
  n = jnp.max(n_vec)
  n_pad = ((n + CH - 1) >> 7) << 7

  first_pos = pos2d_v[0, pl.ds(0, 16)]
  first_row = rows2d_v[0, pl.ds(0, 16)]
  neg = jnp.full((16,), jnp.int32(-2**31))
  pad_pos = jnp.full((16,), jnp.max(jnp.where(lanes == 0, first_pos, neg)))
  pad_row = jnp.full((16,), jnp.max(jnp.where(lanes == 0, first_row, neg)))

  def padfill(t, carry):
    g = t * 16 + lanes
    mp = g >= n
    plsc.store_scatter(pos2d_v, [g >> 7, g & 127], pad_pos, mask=mp)
    plsc.store_scatter(rows2d_v, [g >> 7, g & 127], pad_row, mask=mp)
    return carry

  lax.fori_loop(n >> 4, n_pad >> 4, padfill, 0)

  def phase3(g, carry):
    pltpu.async_copy(val_hbm.at[pos2d_v.at[g]], buf_v, gat_sem).wait()
    pltpu.async_copy(buf_v, out_hbm.at[rows2d_v.at[g]], sct_sem).wait()
    return carry

  lax.fori_loop(0, n_pad >> 7, phase3, 0)


_sc_kernel = functools.partial(
    pl.kernel,
    out_type=(),
    mesh=_mesh,
    compiler_params=pltpu.CompilerParams(needs_layout_passes=False,
                                         use_tc_tiling_on_sc=True),
    scratch_types=[
        pltpu.VMEM((B,), jnp.int32),
        pltpu.VMEM((R + TAIL,), jnp.int32),
        pltpu.VMEM((NSEG, CH), jnp.int32),
        pltpu.VMEM((NSEG, CH), jnp.int32),
        pltpu.VMEM((CH, DP), jnp.float32),
        pltpu.SemaphoreType.DMA,
        pltpu.SemaphoreType.DMA,
    ],
)(_sc_body)


WIN = 32768


def _tin_body(inT_ref, out_ref):
  x = inT_ref[...]
  xt = jnp.swapaxes(x, 0, 1)
  out_ref[...] = jnp.concatenate(
      [xt, jnp.zeros((WIN, DP - D), jnp.float32)], axis=1)


_tin = pl.pallas_call(
    _tin_body,
    grid=((M + WIN - 1) // WIN,),
    in_specs=[pl.BlockSpec((D, WIN), lambda i: (0, i))],
    out_specs=pl.BlockSpec((WIN, DP), lambda i: (i, 0)),
    out_shape=jax.ShapeDtypeStruct((M, DP), jnp.float32),
)


def _tout_body(in_ref, outT_ref):
  outT_ref[...] = jnp.swapaxes(in_ref[:, :D], 0, 1)


_tout = pl.pallas_call(
    _tout_body,
    grid=((M + WIN - 1) // WIN,),
    in_specs=[pl.BlockSpec((WIN, DP), lambda i: (i, 0))],
    out_specs=pl.BlockSpec((D, WIN), lambda i: (0, i)),
    out_shape=jax.ShapeDtypeStruct((D, M), jnp.float32),
)


def kernel(mem, val, idx):
  mem_pad = _tin(mem.T)
  val_pad = jnp.concatenate(
      [val, jnp.zeros((B, DP - D), jnp.float32)], axis=1)
  out_ref = jax.new_ref(mem_pad)
  _sc_kernel(out_ref, val_pad, idx.astype(jnp.int32))
  return _tout(out_ref[...]).T
